# baseline (speedup 1.0000x reference)
import jax
import jax.numpy as jnp
from jax import lax
from jax.experimental import pallas as pl
from jax.experimental.pallas import tpu as pltpu
from jax.experimental.pallas import tpu_sc as plsc

HH, WW, CC = 512, 512, 96
NN, PP = 8, 32768
B = NN * PP

NC, NS, L = 2, 16, 16
NW = NC * NS
PER_W = B // NW
CQ = 64
CHUNKS = PER_W // CQ
NBUF = 3


def _sc_body(table, xs, ys, out, sets, gsems, osems, xsems):
  wid = lax.axis_index("s") * NC + lax.axis_index("c")
  base0 = wid * PER_W

  def fire_xy(t, si):
    xv, yv = sets[si][0], sets[si][1]
    base = base0 + t * CQ
    pltpu.async_copy(xs.at[pl.ds(base, CQ)], xv, xsems[si])
    pltpu.async_copy(ys.at[pl.ds(base, CQ)], yv, xsems[si])

  def stage_fire(t, si):
    xv, yv, ia, ib, ic, idd, wa, wb, wc, wd, ga, gb, gc, gd, ob = sets[si]
    pltpu.make_async_copy(xs.at[pl.ds(0, CQ)], xv, xsems[si]).wait()
    pltpu.make_async_copy(ys.at[pl.ds(0, CQ)], yv, xsems[si]).wait()

    def idx_body(g, c):
      s = pl.ds(g * L, L)
      x = xv[s]
      y = yv[s]
      x0 = x.astype(jnp.int32)
      y0 = y.astype(jnp.int32)
      fx = x - x0.astype(jnp.float32)
      fy = y - y0.astype(jnp.float32)
      i0 = x0 * WW + y0
      ia[s] = i0
      ib[s] = i0 + 1
      ic[s] = i0 + WW
      idd[s] = i0 + (WW + 1)
      gx = 1.0 - fx
      gy = 1.0 - fy
      wa[s] = gx * gy
      wb[s] = gx * fy
      wc[s] = fx * gy
      wd[s] = fx * fy
      return c

    lax.fori_loop(0, CQ // L, idx_body, 0, unroll=2)
    pltpu.async_copy(table.at[ia], ga, gsems[si])
    pltpu.async_copy(table.at[ib], gb, gsems[si])
    pltpu.async_copy(table.at[ic], gc, gsems[si])
    pltpu.async_copy(table.at[idd], gd, gsems[si])

    @pl.when(t + NBUF < CHUNKS)
    def _prefetch_xy():
      fire_xy(t + NBUF, si)

  def wait_gathers(si):
    ia, ga = sets[si][2], sets[si][10]
    for _ in range(4):
      pltpu.make_async_copy(table.at[ia], ga, gsems[si]).wait()

  def compute(t, si):
    _, _, _, _, _, _, wa, wb, wc, wd, ga, gb, gc, gd, ob = sets[si]

    @pl.when(t >= NBUF)
    def _drain_prev_writeback():
      pltpu.make_async_copy(
          ob, out.at[pl.ds(0, CQ), pl.ds(0, CC)], osems[si]).wait()

    def grp_body(g, c):
      s = pl.ds(g * L, L)
      va = wa[s]
      vb = wb[s]
      vc = wc[s]
      vd = wd[s]

      def q_body(q, c2):
        qi = g * L + q
        lane = jnp.full((L,), q, jnp.int32)
        sa = va.at[lane].get(mode="promise_in_bounds")
        sb = vb.at[lane].get(mode="promise_in_bounds")
        sc = vc.at[lane].get(mode="promise_in_bounds")
        sd = vd.at[lane].get(mode="promise_in_bounds")
        for k in range(CC // L):
          sl = pl.ds(k * L, L)
          ob[qi, sl] = (sa * ga[qi, sl] + sb * gb[qi, sl]
                        + sc * gc[qi, sl] + sd * gd[qi, sl])
        return c2

      lax.fori_loop(0, L, q_body, 0, unroll=4)
      return c

    lax.fori_loop(0, CQ // L, grp_body, 0)
    pltpu.async_copy(
        ob, out.at[pl.ds(base0 + t * CQ, CQ), pl.ds(0, CC)], osems[si])

  for s in range(NBUF):
    fire_xy(s, s)
  stage_fire(0, 0)
  stage_fire(1, 1)

  def outer(tt, carry):
    for b in range(NBUF):
      t = tt * NBUF + b

      @pl.when(t + 2 < CHUNKS)
      def _fire_next():
        stage_fire(t + 2, (b + 2) % NBUF)

      wait_gathers(b)
      compute(t, b)
    return carry

  lax.fori_loop(0, CHUNKS // NBUF, outer, 0)
  for t in range(CHUNKS - CHUNKS % NBUF, CHUNKS):
    wait_gathers(t % NBUF)
    compute(t, t % NBUF)
  for b in range(NBUF):
    pltpu.make_async_copy(
        sets[b][14], out.at[pl.ds(0, CQ), pl.ds(0, CC)], osems[b]).wait()


def _one_set():
  return [
      pltpu.VMEM((CQ,), jnp.float32),
      pltpu.VMEM((CQ,), jnp.float32),
      pltpu.VMEM((CQ,), jnp.int32),
      pltpu.VMEM((CQ,), jnp.int32),
      pltpu.VMEM((CQ,), jnp.int32),
      pltpu.VMEM((CQ,), jnp.int32),
      pltpu.VMEM((CQ,), jnp.float32),
      pltpu.VMEM((CQ,), jnp.float32),
      pltpu.VMEM((CQ,), jnp.float32),
      pltpu.VMEM((CQ,), jnp.float32),
      pltpu.VMEM((CQ, 128), jnp.float32),
      pltpu.VMEM((CQ, 128), jnp.float32),
      pltpu.VMEM((CQ, 128), jnp.float32),
      pltpu.VMEM((CQ, 128), jnp.float32),
      pltpu.VMEM((CQ, CC), jnp.float32),
  ]


def _tc_relayout_body(eye_ref, in_ref, out_ref):
  out_ref[:, pl.ds(0, CC)] = jax.lax.dot_general(
      eye_ref[...], in_ref[0], (((1,), (1,)), ((), ())),
      preferred_element_type=jnp.float32)


@jax.jit
def _tc_relayout(data_t):
  eye = jnp.eye(WW, dtype=jnp.float32)
  return pl.pallas_call(
      _tc_relayout_body,
      grid=(HH,),
      in_specs=[
          pl.BlockSpec((WW, WW), lambda h: (0, 0)),
          pl.BlockSpec((1, CC, WW), lambda h: (h, 0, 0)),
      ],
      out_specs=pl.BlockSpec((WW, 128), lambda h: (h, 0)),
      out_shape=jax.ShapeDtypeStruct((HH * WW, 128), jnp.float32),
  )(eye, data_t)


@jax.jit
def _interp(table, xs, ys):
  mesh = plsc.VectorSubcoreMesh(core_axis_name="c", subcore_axis_name="s")
  return pl.kernel(
      _sc_body,
      out_type=jax.ShapeDtypeStruct((B, 128), jnp.float32),
      mesh=mesh,
      compiler_params=pltpu.CompilerParams(use_tc_tiling_on_sc=False),
      scratch_types=[
          [_one_set() for _ in range(NBUF)],
          [pltpu.SemaphoreType.DMA for _ in range(NBUF)],
          [pltpu.SemaphoreType.DMA for _ in range(NBUF)],
          [pltpu.SemaphoreType.DMA for _ in range(NBUF)],
      ],
  )(table, xs, ys)


def kernel(data, sub_x, sub_y):
  table = _tc_relayout(jnp.transpose(data[0], (0, 2, 1)))
  xs = sub_x.reshape(-1)
  ys = sub_y.reshape(-1)
  out = _interp(table, xs, ys)
  return out[:, :CC].reshape(NN, PP, CC)


# speedup vs baseline: 1.7367x; 1.7367x over previous
import jax
import jax.numpy as jnp
from jax import lax
from jax.experimental import pallas as pl
from jax.experimental.pallas import tpu as pltpu
from jax.experimental.pallas import tpu_sc as plsc

HH, WW, CC = 512, 512, 96
NN, PP = 8, 32768
B = NN * PP

NC, NS, L = 2, 16, 16
NW = NC * NS
PER_W = B // NW
CQ = 128
CHUNKS = PER_W // CQ
NBUF = 2


def _sc_body(table, xs, ys, out, sets, gsems, osems, xsems):
  wid = lax.axis_index("s") * NC + lax.axis_index("c")
  base0 = wid * PER_W

  def fire_xy(t, si):
    xv, yv = sets[si][0], sets[si][1]
    base = base0 + t * CQ
    pltpu.async_copy(xs.at[pl.ds(base, CQ)], xv, xsems[si])
    pltpu.async_copy(ys.at[pl.ds(base, CQ)], yv, xsems[si])

  def stage_fire(t, si):
    xv, yv, ia, ib, ic, idd, wa, wb, wc, wd, ga, gb, gc, gd, ob = sets[si]
    pltpu.make_async_copy(xs.at[pl.ds(0, CQ)], xv, xsems[si]).wait()
    pltpu.make_async_copy(ys.at[pl.ds(0, CQ)], yv, xsems[si]).wait()

    def idx_body(g, c):
      s = pl.ds(g * L, L)
      x = xv[s]
      y = yv[s]
      x0 = x.astype(jnp.int32)
      y0 = y.astype(jnp.int32)
      fx = x - x0.astype(jnp.float32)
      fy = y - y0.astype(jnp.float32)
      i0 = x0 * WW + y0
      ia[s] = i0
      ib[s] = i0 + 1
      ic[s] = i0 + WW
      idd[s] = i0 + (WW + 1)
      gx = 1.0 - fx
      gy = 1.0 - fy
      wa[s] = gx * gy
      wb[s] = gx * fy
      wc[s] = fx * gy
      wd[s] = fx * fy
      return c

    lax.fori_loop(0, CQ // L, idx_body, 0, unroll=4)
    pltpu.async_copy(table.at[ia], ga, gsems[si])
    pltpu.async_copy(table.at[ib], gb, gsems[si])
    pltpu.async_copy(table.at[ic], gc, gsems[si])
    pltpu.async_copy(table.at[idd], gd, gsems[si])

    @pl.when(t + NBUF < CHUNKS)
    def _prefetch_xy():
      fire_xy(t + NBUF, si)

  def wait_gathers(si):
    ia, ga = sets[si][2], sets[si][10]
    for _ in range(4):
      pltpu.make_async_copy(table.at[ia], ga, gsems[si]).wait()

  def compute(t, si):
    _, _, _, _, _, _, wa, wb, wc, wd, ga, gb, gc, gd, ob = sets[si]

    @pl.when(t >= NBUF)
    def _drain_prev_writeback():
      pltpu.make_async_copy(
          ob, out.at[pl.ds(0, CQ), pl.ds(0, CC)], osems[si]).wait()

    def grp_body(g, c):
      s = pl.ds(g * L, L)
      va = wa[s]
      vb = wb[s]
      vc = wc[s]
      vd = wd[s]

      def q_body(q, c2):
        qi = g * L + q
        lane = jnp.full((L,), q, jnp.int32)
        sa = va.at[lane].get(mode="promise_in_bounds")
        sb = vb.at[lane].get(mode="promise_in_bounds")
        sc = vc.at[lane].get(mode="promise_in_bounds")
        sd = vd.at[lane].get(mode="promise_in_bounds")
        for k in range(CC // L):
          sl = pl.ds(k * L, L)
          ob[qi, sl] = (sa * ga[qi, sl] + sb * gb[qi, sl]
                        + sc * gc[qi, sl] + sd * gd[qi, sl])
        return c2

      lax.fori_loop(0, L, q_body, 0, unroll=8)
      return c

    lax.fori_loop(0, CQ // L, grp_body, 0)
    pltpu.async_copy(
        ob, out.at[pl.ds(base0 + t * CQ, CQ), pl.ds(0, CC)], osems[si])

  for s in range(NBUF):
    fire_xy(s, s)
  stage_fire(0, 0)

  def outer(tt, carry):
    for b in range(NBUF):
      t = tt * NBUF + b

      @pl.when(t + 1 < CHUNKS)
      def _fire_next():
        stage_fire(t + 1, (b + 1) % NBUF)

      wait_gathers(b)
      compute(t, b)
    return carry

  lax.fori_loop(0, CHUNKS // NBUF, outer, 0)
  for b in range(NBUF):
    pltpu.make_async_copy(
        sets[b][14], out.at[pl.ds(0, CQ), pl.ds(0, CC)], osems[b]).wait()


def _one_set():
  return [
      pltpu.VMEM((CQ,), jnp.float32),
      pltpu.VMEM((CQ,), jnp.float32),
      pltpu.VMEM((CQ,), jnp.int32),
      pltpu.VMEM((CQ,), jnp.int32),
      pltpu.VMEM((CQ,), jnp.int32),
      pltpu.VMEM((CQ,), jnp.int32),
      pltpu.VMEM((CQ,), jnp.float32),
      pltpu.VMEM((CQ,), jnp.float32),
      pltpu.VMEM((CQ,), jnp.float32),
      pltpu.VMEM((CQ,), jnp.float32),
      pltpu.VMEM((CQ, CC), jnp.float32),
      pltpu.VMEM((CQ, CC), jnp.float32),
      pltpu.VMEM((CQ, CC), jnp.float32),
      pltpu.VMEM((CQ, CC), jnp.float32),
      pltpu.VMEM((CQ, CC), jnp.float32),
  ]


@jax.jit
def _interp(table, xs, ys):
  mesh = plsc.VectorSubcoreMesh(core_axis_name="c", subcore_axis_name="s")
  return pl.kernel(
      _sc_body,
      out_type=jax.ShapeDtypeStruct((B, 128), jnp.float32),
      mesh=mesh,
      compiler_params=pltpu.CompilerParams(use_tc_tiling_on_sc=False),
      scratch_types=[
          [_one_set() for _ in range(NBUF)],
          [pltpu.SemaphoreType.DMA for _ in range(NBUF)],
          [pltpu.SemaphoreType.DMA for _ in range(NBUF)],
          [pltpu.SemaphoreType.DMA for _ in range(NBUF)],
      ],
  )(table, xs, ys)


def kernel(data, sub_x, sub_y):
  table = data.reshape(HH * WW, CC)
  xs = sub_x.reshape(-1)
  ys = sub_y.reshape(-1)
  out = _interp(table, xs, ys)
  return out[:, :CC].reshape(NN, PP, CC)


# speedup vs baseline: 1.7401x; 1.0020x over previous
import jax
import jax.numpy as jnp
from jax import lax
from jax.experimental import pallas as pl
from jax.experimental.pallas import tpu as pltpu
from jax.experimental.pallas import tpu_sc as plsc

HH, WW, CC = 512, 512, 96
NN, PP = 8, 32768
B = NN * PP

NC, NS, L = 2, 16, 16
NW = NC * NS
PER_W = B // NW
CQ = 128
CHUNKS = PER_W // CQ
NBUF = 2


def _sc_body(table, xs, ys, out, sets, gsems, osems, xsems):
  wid = lax.axis_index("s") * NC + lax.axis_index("c")
  base0 = wid * PER_W

  def fire_xy(t, si):
    xv, yv = sets[si][0], sets[si][1]
    base = base0 + t * CQ
    pltpu.async_copy(xs.at[pl.ds(base, CQ)], xv, xsems[si])
    pltpu.async_copy(ys.at[pl.ds(base, CQ)], yv, xsems[si])

  def stage_fire(t, si):
    xv, yv, ia, ib, ic, idd, wa, wb, wc, wd, ga, gb, gc, gd, ob = sets[si]
    pltpu.make_async_copy(xs.at[pl.ds(0, CQ)], xv, xsems[si]).wait()
    pltpu.make_async_copy(ys.at[pl.ds(0, CQ)], yv, xsems[si]).wait()

    def idx_body(g, c):
      s = pl.ds(g * L, L)
      x = xv[s]
      y = yv[s]
      x0 = x.astype(jnp.int32)
      y0 = y.astype(jnp.int32)
      fx = x - x0.astype(jnp.float32)
      fy = y - y0.astype(jnp.float32)
      i0 = x0 * WW + y0
      ia[s] = i0
      ib[s] = i0 + 1
      ic[s] = i0 + WW
      idd[s] = i0 + (WW + 1)
      gx = 1.0 - fx
      gy = 1.0 - fy
      wa[s] = gx * gy
      wb[s] = gx * fy
      wc[s] = fx * gy
      wd[s] = fx * fy
      return c

    lax.fori_loop(0, CQ // L, idx_body, 0, unroll=4)
    pltpu.async_copy(table.at[ia], ga, gsems[si])
    pltpu.async_copy(table.at[ib], gb, gsems[si])
    pltpu.async_copy(table.at[ic], gc, gsems[si])
    pltpu.async_copy(table.at[idd], gd, gsems[si])

    @pl.when(t + NBUF < CHUNKS)
    def _prefetch_xy():
      fire_xy(t + NBUF, si)

  def wait_gathers(si):
    ia, ga = sets[si][2], sets[si][10]
    for _ in range(4):
      pltpu.make_async_copy(table.at[ia], ga, gsems[si]).wait()

  def compute(t, si):
    _, _, _, _, _, _, wa, wb, wc, wd, ga, gb, gc, gd, ob = sets[si]

    @pl.when(t >= NBUF)
    def _drain_prev_writeback():
      pltpu.make_async_copy(
          ob, out.at[pl.ds(0, CQ), pl.ds(0, CC)], osems[si]).wait()

    def grp_body(g, c):
      s = pl.ds(g * L, L)
      va = wa[s]
      vb = wb[s]
      vc = wc[s]
      vd = wd[s]

      def q_body(q, c2):
        qi = g * L + q
        lane = jnp.full((L,), q, jnp.int32)
        sa = va.at[lane].get(mode="promise_in_bounds")
        sb = vb.at[lane].get(mode="promise_in_bounds")
        sc = vc.at[lane].get(mode="promise_in_bounds")
        sd = vd.at[lane].get(mode="promise_in_bounds")
        for k in range(CC // L):
          sl = pl.ds(k * L, L)
          ob[qi, sl] = (sa * ga[qi, sl] + sb * gb[qi, sl]
                        + sc * gc[qi, sl] + sd * gd[qi, sl])
        return c2

      lax.fori_loop(0, L, q_body, 0, unroll=8)
      return c

    lax.fori_loop(0, CQ // L, grp_body, 0, unroll=2)
    pltpu.async_copy(
        ob, out.at[pl.ds(base0 + t * CQ, CQ), pl.ds(0, CC)], osems[si])

  for s in range(NBUF):
    fire_xy(s, s)
  stage_fire(0, 0)

  def outer(tt, carry):
    for b in range(NBUF):
      t = tt * NBUF + b

      @pl.when(t + 1 < CHUNKS)
      def _fire_next():
        stage_fire(t + 1, (b + 1) % NBUF)

      wait_gathers(b)
      compute(t, b)
    return carry

  lax.fori_loop(0, CHUNKS // NBUF, outer, 0)
  for b in range(NBUF):
    pltpu.make_async_copy(
        sets[b][14], out.at[pl.ds(0, CQ), pl.ds(0, CC)], osems[b]).wait()


def _one_set():
  return [
      pltpu.VMEM((CQ,), jnp.float32),
      pltpu.VMEM((CQ,), jnp.float32),
      pltpu.VMEM((CQ,), jnp.int32),
      pltpu.VMEM((CQ,), jnp.int32),
      pltpu.VMEM((CQ,), jnp.int32),
      pltpu.VMEM((CQ,), jnp.int32),
      pltpu.VMEM((CQ,), jnp.float32),
      pltpu.VMEM((CQ,), jnp.float32),
      pltpu.VMEM((CQ,), jnp.float32),
      pltpu.VMEM((CQ,), jnp.float32),
      pltpu.VMEM((CQ, CC), jnp.float32),
      pltpu.VMEM((CQ, CC), jnp.float32),
      pltpu.VMEM((CQ, CC), jnp.float32),
      pltpu.VMEM((CQ, CC), jnp.float32),
      pltpu.VMEM((CQ, CC), jnp.float32),
  ]


@jax.jit
def _interp(table, xs, ys):
  mesh = plsc.VectorSubcoreMesh(core_axis_name="c", subcore_axis_name="s")
  return pl.kernel(
      _sc_body,
      out_type=jax.ShapeDtypeStruct((B, 128), jnp.float32),
      mesh=mesh,
      compiler_params=pltpu.CompilerParams(use_tc_tiling_on_sc=False),
      scratch_types=[
          [_one_set() for _ in range(NBUF)],
          [pltpu.SemaphoreType.DMA for _ in range(NBUF)],
          [pltpu.SemaphoreType.DMA for _ in range(NBUF)],
          [pltpu.SemaphoreType.DMA for _ in range(NBUF)],
      ],
  )(table, xs, ys)


def kernel(data, sub_x, sub_y):
  table = data.reshape(HH * WW, CC)
  xs = sub_x.reshape(-1)
  ys = sub_y.reshape(-1)
  out = _interp(table, xs, ys)
  return out[:, :CC].reshape(NN, PP, CC)
